# Initial kernel scaffold; baseline (speedup 1.0000x reference)
import jax
import jax.numpy as jnp
from jax.experimental import pallas as pl


def kernel(node_s, node_v, edge_index, edge_s, edge_v, batch, params):
    raise NotImplementedError("write your pallas kernel here")


# speedup vs baseline: 3.8487x; 3.8487x over previous
import jax
import jax.numpy as jnp
from jax.experimental import pallas as pl
from jax.experimental.pallas import tpu as pltpu

_EPS = 1e-8
_E_BLOCK = 1600


def _norm3(vx, vy, vz):
    return jnp.sqrt(jnp.maximum(vx * vx + vy * vy + vz * vz, _EPS))


def _dot(a, b):
    return jnp.dot(a, b, preferred_element_type=jnp.float32)


def _msg_kernel(
    s_src_ref, s_dst_ref, vx_src_ref, vy_src_ref, vz_src_ref,
    vx_dst_ref, vy_dst_ref, vz_dst_ref,
    eraw_ref,
    ee_whc_ref, ee_ws0_ref, ee_wsT_ref, ee_wsb_ref, ee_wvT_ref,
    m0_whT_ref, m0_wsT_ref, m0_wsb_ref, m0_wvT_ref,
    m1_whT_ref, m1_wsT_ref, m1_wsb_ref, m1_wvT_ref,
    m2_whT_ref, m2_wsT_ref, m2_wsb_ref, m2_wvT_ref,
    ms_ref, mvx_ref, mvy_ref, mvz_ref,
):
    e_s = eraw_ref[:, 0:1]
    whc = ee_whc_ref[0:1, :]
    vhx = eraw_ref[:, 1:2] * whc
    vhy = eraw_ref[:, 2:3] * whc
    vhz = eraw_ref[:, 3:4] * whc
    vn = _norm3(vhx, vhy, vhz)
    es = e_s * ee_ws0_ref[0:1, :] + _dot(vn, ee_wsT_ref[...]) + ee_wsb_ref[0:1, :]
    es = jnp.maximum(es, 0.0)
    evx = _dot(vhx, ee_wvT_ref[...])
    evy = _dot(vhy, ee_wvT_ref[...])
    evz = _dot(vhz, ee_wvT_ref[...])
    sg = jax.nn.sigmoid(_norm3(evx, evy, evz))
    evx, evy, evz = evx * sg, evy * sg, evz * sg

    whT = m0_whT_ref
    wA = whT[0:64, :]
    wB = whT[64:128, :]
    wC = whT[128:192, :]
    vhx = _dot(vx_src_ref[...], wA) + _dot(evx, wB) + _dot(vx_dst_ref[...], wC)
    vhy = _dot(vy_src_ref[...], wA) + _dot(evy, wB) + _dot(vy_dst_ref[...], wC)
    vhz = _dot(vz_src_ref[...], wA) + _dot(evz, wB) + _dot(vz_dst_ref[...], wC)
    vn = _norm3(vhx, vhy, vhz)
    wsT = m0_wsT_ref
    s = (_dot(s_src_ref[...], wsT[0:64, :]) + _dot(es, wsT[64:128, :])
         + _dot(s_dst_ref[...], wsT[128:192, :]) + _dot(vn, wsT[192:384, :])
         + m0_wsb_ref[0:1, :])
    vox = _dot(vhx, m0_wvT_ref[...])
    voy = _dot(vhy, m0_wvT_ref[...])
    voz = _dot(vhz, m0_wvT_ref[...])
    sg = jax.nn.sigmoid(_norm3(vox, voy, voz))
    vx, vy, vz = vox * sg, voy * sg, voz * sg
    s = jnp.maximum(s, 0.0)

    def gvp64(s, vx, vy, vz, whT, wsT, wsb, wvT, act):
        vhx = _dot(vx, whT[...])
        vhy = _dot(vy, whT[...])
        vhz = _dot(vz, whT[...])
        vn = _norm3(vhx, vhy, vhz)
        s_out = _dot(s, wsT[0:64, :]) + _dot(vn, wsT[64:128, :]) + wsb[0:1, :]
        vox = _dot(vhx, wvT[...])
        voy = _dot(vhy, wvT[...])
        voz = _dot(vhz, wvT[...])
        if act:
            sg = jax.nn.sigmoid(_norm3(vox, voy, voz))
            vox, voy, voz = vox * sg, voy * sg, voz * sg
            s_out = jnp.maximum(s_out, 0.0)
        return s_out, vox, voy, voz

    s, vx, vy, vz = gvp64(s, vx, vy, vz, m1_whT_ref, m1_wsT_ref, m1_wsb_ref, m1_wvT_ref, True)
    s, vx, vy, vz = gvp64(s, vx, vy, vz, m2_whT_ref, m2_wsT_ref, m2_wsb_ref, m2_wvT_ref, False)

    ms_ref[...] = s
    mvx_ref[...] = vx
    mvy_ref[...] = vy
    mvz_ref[...] = vz


def _edge_messages(sg_src, sg_dst, vgx_src, vgy_src, vgz_src,
                   vgx_dst, vgy_dst, vgz_dst, eraw, pe, p0, p1, p2):
    E = sg_src.shape[0]
    B = _E_BLOCK
    grid = (E // B,)
    f32 = jnp.float32

    def rowspec(c):
        return pl.BlockSpec((B, c), lambda i: (i, 0))

    def wspec(shape):
        return pl.BlockSpec(shape, lambda i: tuple(0 for _ in shape))

    ee_whc = pe["wh"].T.reshape(1, 64)
    ee_ws0 = pe["ws"].T[0:1, :]
    ee_wsT = pe["ws"].T[1:, :]
    ee_wsb = pe["ws_b"].reshape(1, 64)
    ee_wvT = pe["wv"].T

    def prep(p):
        return (p["wh"].T, p["ws"].T, p["ws_b"].reshape(1, -1), p["wv"].T)

    m0, m1, m2 = prep(p0), prep(p1), prep(p2)

    in_arrays = [
        sg_src, sg_dst, vgx_src, vgy_src, vgz_src, vgx_dst, vgy_dst, vgz_dst,
        eraw,
        ee_whc, ee_ws0, ee_wsT, ee_wsb, ee_wvT,
        *m0, *m1, *m2,
    ]
    in_specs = [
        rowspec(64), rowspec(64),
        rowspec(64), rowspec(64), rowspec(64),
        rowspec(64), rowspec(64), rowspec(64),
        rowspec(4),
        wspec((1, 64)), wspec((1, 64)), wspec((64, 64)), wspec((1, 64)), wspec((64, 64)),
        wspec((192, 192)), wspec((384, 64)), wspec((1, 64)), wspec((192, 64)),
        wspec((64, 64)), wspec((128, 64)), wspec((1, 64)), wspec((64, 64)),
        wspec((64, 64)), wspec((128, 64)), wspec((1, 64)), wspec((64, 64)),
    ]
    out_shape = [jax.ShapeDtypeStruct((E, 64), f32) for _ in range(4)]
    out_specs = [rowspec(64)] * 4
    return pl.pallas_call(
        _msg_kernel,
        grid=grid,
        in_specs=in_specs,
        out_specs=out_specs,
        out_shape=out_shape,
    )(*in_arrays)


def _gvp_t(p, s, vt, scalar_act, vector_act):
    vh = jnp.matmul(vt, p["wh"].T)
    vn = jnp.sqrt(jnp.maximum(jnp.sum(vh * vh, axis=-2), _EPS))
    s_out = jnp.matmul(jnp.concatenate([s, vn], axis=-1), p["ws"].T) + p["ws_b"]
    vt_out = jnp.matmul(vh, p["wv"].T)
    if vector_act:
        nrm = jnp.sqrt(jnp.maximum(jnp.sum(vt_out * vt_out, axis=-2, keepdims=True), _EPS))
        vt_out = vt_out * jax.nn.sigmoid(nrm)
    if scalar_act:
        s_out = jax.nn.relu(s_out)
    return s_out, vt_out


def _layernorm_s(g, b, s):
    mu = jnp.mean(s, axis=-1, keepdims=True)
    var = jnp.mean(jnp.square(s - mu), axis=-1, keepdims=True)
    return (s - mu) / jnp.sqrt(var + 1e-5) * g + b


def kernel(node_s, node_v, edge_index, edge_s, edge_v, batch, params):
    n = node_s.shape[0]
    num_graphs = 32
    if node_v.ndim == 2:
        node_v = node_v[:, None, :]
    if edge_v.ndim == 2:
        edge_v = edge_v[:, None, :]
    node_vt = jnp.swapaxes(node_v, -1, -2)
    ns, nvt = _gvp_t(params["node_enc"], node_s, node_vt, True, True)

    p = params["convs"][-1]
    src, dst = edge_index[0], edge_index[1]

    tx = nvt[:, 0, :]
    ty = nvt[:, 1, :]
    tz = nvt[:, 2, :]
    eraw = jnp.concatenate([edge_s, edge_v[:, 0, :]], axis=-1)

    ms, mvx, mvy, mvz = _edge_messages(
        ns[src], ns[dst], tx[src], ty[src], tz[src], tx[dst], ty[dst], tz[dst],
        eraw, params["edge_enc"], p["msg0"], p["msg1"], p["msg2"])

    cnt = jnp.maximum(jax.ops.segment_sum(jnp.ones((dst.shape[0],), jnp.float32), dst, num_segments=n), 1.0)
    agg_s = jax.ops.segment_sum(ms, dst, num_segments=n) / cnt[:, None]
    aggx = jax.ops.segment_sum(mvx, dst, num_segments=n) / cnt[:, None]
    aggy = jax.ops.segment_sum(mvy, dst, num_segments=n) / cnt[:, None]
    aggz = jax.ops.segment_sum(mvz, dst, num_segments=n) / cnt[:, None]
    agg_v = jnp.stack([aggx, aggy, aggz], axis=1)

    s1 = _layernorm_s(p["ln0_g"], p["ln0_b"], ns + agg_s)
    v1 = nvt + agg_v
    vn = jnp.maximum(jnp.sum(v1 * v1, axis=-2, keepdims=True), _EPS)
    vn = jnp.sqrt(jnp.mean(vn, axis=-1, keepdims=True))
    v1 = v1 / vn

    fs, fvt = _gvp_t(p["ff0"], s1, v1, True, True)
    fs, _ = _gvp_t(p["ff1"], fs, fvt, False, False)
    xs = _layernorm_s(p["ln1_g"], p["ln1_b"], s1 + fs)

    cntb = jnp.maximum(jax.ops.segment_sum(jnp.ones((n,), jnp.float32), batch, num_segments=num_graphs), 1.0)
    pooled = jax.ops.segment_sum(xs, batch, num_segments=num_graphs) / cntb[:, None]
    w0, b0 = params["mlp"][0]
    w1, b1 = params["mlp"][1]
    w2, b2 = params["mlp"][2]
    h = jax.nn.relu(jnp.matmul(pooled, w0.T) + b0)
    h = jax.nn.relu(jnp.matmul(h, w1.T) + b1)
    return jax.nn.sigmoid(jnp.matmul(h, w2.T) + b2)


# speedup vs baseline: 12.1444x; 3.1555x over previous
import functools

import jax
import jax.numpy as jnp
from jax import lax
from jax.experimental import pallas as pl
from jax.experimental.pallas import tpu as pltpu
from jax.experimental.pallas import tpu_sc as plsc

_EPS = 1e-8
_E_BLOCK = 1600
_CH = 128
_N_NODES = 10000
_NC, _NS = 2, 16


def _norm3(vx, vy, vz):
    return jnp.sqrt(jnp.maximum(vx * vx + vy * vy + vz * vz, _EPS))


def _dot(a, b):
    return jnp.dot(a, b, preferred_element_type=jnp.float32)


def _sc_gather(t_sx, t_yz, idx):
    rows = idx.shape[0]
    n_chunks = rows // _CH
    nw = _NC * _NS
    per, extra = divmod(n_chunks, nw)
    f32 = jnp.float32
    mesh = plsc.VectorSubcoreMesh(core_axis_name="c", subcore_axis_name="s")

    @functools.partial(
        pl.kernel,
        out_type=jax.ShapeDtypeStruct((2, rows, 128), f32),
        mesh=mesh,
        scratch_types=[
            pltpu.VMEM((_CH,), jnp.int32),
            pltpu.VMEM((_CH, 128), f32),
            pltpu.VMEM((_CH, 128), f32),
            pltpu.SemaphoreType.DMA,
            pltpu.SemaphoreType.DMA,
        ],
    )
    def k(t_sx_h, t_yz_h, idx_h, out_h,
          idx_v, b0, b1, s0, s1):
        c = lax.axis_index("c")
        s = lax.axis_index("s")
        w = s * _NC + c
        start = per * w + jnp.minimum(w, extra)
        count = per + (w < extra).astype(jnp.int32)

        def body(i, carry):
            t = start + i
            r = pl.multiple_of(t * _CH, _CH)
            pltpu.sync_copy(idx_h.at[pl.ds(r, _CH)], idx_v)
            cp0 = pltpu.async_copy(t_sx_h.at[idx_v], b0, s0)
            cp1 = pltpu.async_copy(t_yz_h.at[idx_v], b1, s1)
            cp0.wait()
            cp1.wait()
            pltpu.sync_copy(b0, out_h.at[0, pl.ds(r, _CH)])
            pltpu.sync_copy(b1, out_h.at[1, pl.ds(r, _CH)])
            return carry

        lax.fori_loop(0, count, body, 0)

    return k(t_sx, t_yz, idx)


def _sc_scatter(msgs, dst):
    E = dst.shape[0]
    n_chunks = E // _CH
    stripe = 632
    n_pad = stripe * _NS
    f32 = jnp.float32
    per, extra = divmod(n_chunks, _NS)
    mesh = plsc.VectorSubcoreMesh(core_axis_name="c", subcore_axis_name="s")

    @functools.partial(
        pl.kernel,
        out_type=jax.ShapeDtypeStruct((2 * n_pad, 128), f32),
        mesh=mesh,
        scratch_types=[
            pltpu.VMEM((_CH,), jnp.int32),
            pltpu.VMEM((_CH, 128), f32),
            pltpu.VMEM_SHARED((n_pad, 128), f32),
        ],
    )
    def k(msgs_h, dst_h, zeros_h, agg_h,
          idx_v, ba, acc):
        c = lax.axis_index("c")
        s = lax.axis_index("s")

        base = pl.multiple_of(s * stripe, 8)
        start = per * s + jnp.minimum(s, extra)
        count = per + (s < extra).astype(jnp.int32)
        row0 = c * E

        pltpu.sync_copy(zeros_h, acc.at[pl.ds(base, stripe)])
        plsc.subcore_barrier()

        def body(i, carry):
            t = start + i
            r = pl.multiple_of(t * _CH, _CH)
            pltpu.sync_copy(dst_h.at[pl.ds(r, _CH)], idx_v)
            pltpu.sync_copy(msgs_h.at[pl.ds(pl.multiple_of(row0 + r, _CH), _CH)], ba)
            pltpu.sync_copy(ba, acc.at[idx_v], add=True)
            return carry

        lax.fori_loop(0, count, body, 0)
        plsc.subcore_barrier()

        out_r = pl.multiple_of(c * n_pad + base, 8)
        pltpu.sync_copy(acc.at[pl.ds(base, stripe)],
                        agg_h.at[pl.ds(out_r, stripe)])

    agg = k(msgs.reshape(2 * E, 128), dst, jnp.zeros((stripe, 128), f32))
    return agg.reshape(2, n_pad, 128)[:, :_N_NODES, :]


def _msg_kernel(
    g0s_ref, g0d_ref, g1s_ref, g1d_ref,
    eraw_ref,
    ee_whc_ref, ee_ws0_ref, ee_wsT_ref, ee_wsb_ref, ee_wvT_ref,
    m0_whT_ref, m0_wsT_ref, m0_wsb_ref, m0_wvT_ref,
    m1_whT_ref, m1_wsT_ref, m1_wsb_ref, m1_wvT_ref,
    m2_whT_ref, m2_wsT_ref, m2_wsb_ref, m2_wvT_ref,
    out_ref,
):
    e_s = eraw_ref[:, 0:1]
    whc = ee_whc_ref[0:1, :]
    vhx = eraw_ref[:, 1:2] * whc
    vhy = eraw_ref[:, 2:3] * whc
    vhz = eraw_ref[:, 3:4] * whc
    vn = _norm3(vhx, vhy, vhz)
    es = e_s * ee_ws0_ref[0:1, :] + _dot(vn, ee_wsT_ref[...]) + ee_wsb_ref[0:1, :]
    es = jnp.maximum(es, 0.0)
    evx = _dot(vhx, ee_wvT_ref[...])
    evy = _dot(vhy, ee_wvT_ref[...])
    evz = _dot(vhz, ee_wvT_ref[...])
    sg = jax.nn.sigmoid(_norm3(evx, evy, evz))
    evx, evy, evz = evx * sg, evy * sg, evz * sg

    g0s = g0s_ref[0]
    g0d = g0d_ref[0]
    g1s = g1s_ref[0]
    g1d = g1d_ref[0]
    s_src, vx_src = g0s[:, 0:64], g0s[:, 64:128]
    s_dst, vx_dst = g0d[:, 0:64], g0d[:, 64:128]
    vy_src, vz_src = g1s[:, 0:64], g1s[:, 64:128]
    vy_dst, vz_dst = g1d[:, 0:64], g1d[:, 64:128]
    whT = m0_whT_ref
    wA = whT[0:64, :]
    wB = whT[64:128, :]
    wC = whT[128:192, :]
    vhx = _dot(vx_src, wA) + _dot(evx, wB) + _dot(vx_dst, wC)
    vhy = _dot(vy_src, wA) + _dot(evy, wB) + _dot(vy_dst, wC)
    vhz = _dot(vz_src, wA) + _dot(evz, wB) + _dot(vz_dst, wC)
    vn = _norm3(vhx, vhy, vhz)
    wsT = m0_wsT_ref
    s = (_dot(s_src, wsT[0:64, :]) + _dot(es, wsT[64:128, :])
         + _dot(s_dst, wsT[128:192, :]) + _dot(vn, wsT[192:384, :])
         + m0_wsb_ref[0:1, :])
    vox = _dot(vhx, m0_wvT_ref[...])
    voy = _dot(vhy, m0_wvT_ref[...])
    voz = _dot(vhz, m0_wvT_ref[...])
    sg = jax.nn.sigmoid(_norm3(vox, voy, voz))
    vx, vy, vz = vox * sg, voy * sg, voz * sg
    s = jnp.maximum(s, 0.0)

    def gvp64(s, vx, vy, vz, whT, wsT, wsb, wvT, act):
        vhx = _dot(vx, whT[...])
        vhy = _dot(vy, whT[...])
        vhz = _dot(vz, whT[...])
        vn = _norm3(vhx, vhy, vhz)
        s_out = _dot(s, wsT[0:64, :]) + _dot(vn, wsT[64:128, :]) + wsb[0:1, :]
        vox = _dot(vhx, wvT[...])
        voy = _dot(vhy, wvT[...])
        voz = _dot(vhz, wvT[...])
        if act:
            sg = jax.nn.sigmoid(_norm3(vox, voy, voz))
            vox, voy, voz = vox * sg, voy * sg, voz * sg
            s_out = jnp.maximum(s_out, 0.0)
        return s_out, vox, voy, voz

    s, vx, vy, vz = gvp64(s, vx, vy, vz, m1_whT_ref, m1_wsT_ref, m1_wsb_ref, m1_wvT_ref, True)
    s, vx, vy, vz = gvp64(s, vx, vy, vz, m2_whT_ref, m2_wsT_ref, m2_wsb_ref, m2_wvT_ref, False)

    out_ref[0, :, 0:64] = s
    out_ref[0, :, 64:128] = vx
    out_ref[1, :, 0:64] = vy
    out_ref[1, :, 64:128] = vz


def _edge_messages(g, eraw, pe, p0, p1, p2):
    E = eraw.shape[0]
    B = _E_BLOCK
    nb = E // B
    grid = (nb,)
    f32 = jnp.float32

    def gspec(plane, is_dst):
        return pl.BlockSpec(
            (1, B, 128),
            functools.partial(lambda p, o, i: (p, i + o, 0), plane, nb if is_dst else 0))

    def wspec(shape):
        return pl.BlockSpec(shape, lambda i: tuple(0 for _ in shape))

    ee_whc = pe["wh"].T.reshape(1, 64)
    ee_ws0 = pe["ws"].T[0:1, :]
    ee_wsT = pe["ws"].T[1:, :]
    ee_wsb = pe["ws_b"].reshape(1, 64)
    ee_wvT = pe["wv"].T

    def prep(p):
        return (p["wh"].T, p["ws"].T, p["ws_b"].reshape(1, -1), p["wv"].T)

    m0, m1, m2 = prep(p0), prep(p1), prep(p2)

    in_arrays = [
        g, g, g, g,
        eraw,
        ee_whc, ee_ws0, ee_wsT, ee_wsb, ee_wvT,
        *m0, *m1, *m2,
    ]
    in_specs = [
        gspec(0, False), gspec(0, True),
        gspec(1, False), gspec(1, True),
        pl.BlockSpec((B, 4), lambda i: (i, 0)),
        wspec((1, 64)), wspec((1, 64)), wspec((64, 64)), wspec((1, 64)), wspec((64, 64)),
        wspec((192, 192)), wspec((384, 64)), wspec((1, 64)), wspec((192, 64)),
        wspec((64, 64)), wspec((128, 64)), wspec((1, 64)), wspec((64, 64)),
        wspec((64, 64)), wspec((128, 64)), wspec((1, 64)), wspec((64, 64)),
    ]
    return pl.pallas_call(
        _msg_kernel,
        grid=grid,
        in_specs=in_specs,
        out_specs=pl.BlockSpec((2, B, 128), lambda i: (0, i, 0)),
        out_shape=jax.ShapeDtypeStruct((2, E, 128), f32),
    )(*in_arrays)


def _gvp_t(p, s, vt, scalar_act, vector_act):
    vh = jnp.matmul(vt, p["wh"].T)
    vn = jnp.sqrt(jnp.maximum(jnp.sum(vh * vh, axis=-2), _EPS))
    s_out = jnp.matmul(jnp.concatenate([s, vn], axis=-1), p["ws"].T) + p["ws_b"]
    vt_out = jnp.matmul(vh, p["wv"].T)
    if vector_act:
        nrm = jnp.sqrt(jnp.maximum(jnp.sum(vt_out * vt_out, axis=-2, keepdims=True), _EPS))
        vt_out = vt_out * jax.nn.sigmoid(nrm)
    if scalar_act:
        s_out = jax.nn.relu(s_out)
    return s_out, vt_out


def _layernorm_s(g, b, s):
    mu = jnp.mean(s, axis=-1, keepdims=True)
    var = jnp.mean(jnp.square(s - mu), axis=-1, keepdims=True)
    return (s - mu) / jnp.sqrt(var + 1e-5) * g + b


def kernel(node_s, node_v, edge_index, edge_s, edge_v, batch, params):
    n = node_s.shape[0]
    num_graphs = 32
    if node_v.ndim == 2:
        node_v = node_v[:, None, :]
    if edge_v.ndim == 2:
        edge_v = edge_v[:, None, :]
    node_vt = jnp.swapaxes(node_v, -1, -2)
    ns, nvt = _gvp_t(params["node_enc"], node_s, node_vt, True, True)

    p = params["convs"][-1]
    E = edge_index.shape[1]

    t_sx = jnp.concatenate([ns, nvt[:, 0, :]], axis=1)
    t_yz = jnp.concatenate([nvt[:, 1, :], nvt[:, 2, :]], axis=1)
    eraw = jnp.concatenate([edge_s, edge_v[:, 0, :]], axis=-1)

    idx = edge_index.reshape(2 * E)
    g = _sc_gather(t_sx, t_yz, idx)

    msgs = _edge_messages(g, eraw, params["edge_enc"], p["msg0"], p["msg1"], p["msg2"])

    dst = edge_index[1]
    agg = _sc_scatter(msgs, dst)
    cnt = jnp.maximum(
        jax.ops.segment_sum(jnp.ones((E,), jnp.float32), dst, num_segments=n), 1.0)
    agg_s = agg[0, :, 0:64] / cnt[:, None]
    agg_v = jnp.stack([agg[0, :, 64:128], agg[1, :, 0:64], agg[1, :, 64:128]],
                      axis=1) / cnt[:, None, None]

    s1 = _layernorm_s(p["ln0_g"], p["ln0_b"], ns + agg_s)
    v1 = nvt + agg_v
    vn = jnp.maximum(jnp.sum(v1 * v1, axis=-2, keepdims=True), _EPS)
    vn = jnp.sqrt(jnp.mean(vn, axis=-1, keepdims=True))
    v1 = v1 / vn

    fs, fvt = _gvp_t(p["ff0"], s1, v1, True, True)
    fs, _ = _gvp_t(p["ff1"], fs, fvt, False, False)
    xs = _layernorm_s(p["ln1_g"], p["ln1_b"], s1 + fs)

    cntb = jnp.maximum(jax.ops.segment_sum(jnp.ones((n,), jnp.float32), batch, num_segments=num_graphs), 1.0)
    pooled = jax.ops.segment_sum(xs, batch, num_segments=num_graphs) / cntb[:, None]
    w0, b0 = params["mlp"][0]
    w1, b1 = params["mlp"][1]
    w2, b2 = params["mlp"][2]
    h = jax.nn.relu(jnp.matmul(pooled, w0.T) + b0)
    h = jax.nn.relu(jnp.matmul(h, w1.T) + b1)
    return jax.nn.sigmoid(jnp.matmul(h, w2.T) + b2)


# speedup vs baseline: 12.6276x; 1.0398x over previous
import functools

import jax
import jax.numpy as jnp
from jax import lax
from jax.experimental import pallas as pl
from jax.experimental.pallas import tpu as pltpu
from jax.experimental.pallas import tpu_sc as plsc

_EPS = 1e-8
_E_BLOCK = 1600
_CH = 128
_N_NODES = 10000
_NC, _NS = 2, 16


def _norm3(vx, vy, vz):
    return jnp.sqrt(jnp.maximum(vx * vx + vy * vy + vz * vz, _EPS))


def _dot(a, b):
    return jnp.dot(a, b, preferred_element_type=jnp.float32)


def _sc_gather(table, idx):
    rows = idx.shape[0]
    n_chunks = rows // _CH
    nw = _NC * _NS
    per, extra = divmod(n_chunks, nw)
    f32 = jnp.float32
    mesh = plsc.VectorSubcoreMesh(core_axis_name="c", subcore_axis_name="s")

    @functools.partial(
        pl.kernel,
        out_type=jax.ShapeDtypeStruct((rows, 128), f32),
        mesh=mesh,
        scratch_types=[
            pltpu.VMEM((_CH,), jnp.int32),
            pltpu.VMEM((_CH, 128), f32),
            pltpu.SemaphoreType.DMA,
        ],
    )
    def k(table_h, idx_h, out_h,
          idx_v, b0, s0):
        c = lax.axis_index("c")
        s = lax.axis_index("s")
        w = s * _NC + c
        start = per * w + jnp.minimum(w, extra)
        count = per + (w < extra).astype(jnp.int32)

        def body(i, carry):
            t = start + i
            r = pl.multiple_of(t * _CH, _CH)
            pltpu.sync_copy(idx_h.at[pl.ds(r, _CH)], idx_v)
            pltpu.async_copy(table_h.at[idx_v], b0, s0).wait()
            pltpu.sync_copy(b0, out_h.at[pl.ds(r, _CH)])
            return carry

        lax.fori_loop(0, count, body, 0)

    return k(table, idx)


def _sc_scatter(msgs, dst):
    E = dst.shape[0]
    n_chunks = E // _CH
    stripe = 632
    n_pad = stripe * _NS
    f32 = jnp.float32
    per, extra = divmod(n_chunks, _NS)
    mesh = plsc.VectorSubcoreMesh(core_axis_name="c", subcore_axis_name="s")

    @functools.partial(
        pl.kernel,
        out_type=jax.ShapeDtypeStruct((2 * n_pad, 128), f32),
        mesh=mesh,
        scratch_types=[
            pltpu.VMEM((_CH,), jnp.int32),
            pltpu.VMEM((_CH, 128), f32),
            pltpu.VMEM_SHARED((n_pad, 128), f32),
        ],
    )
    def k(msgs_h, dst_h, zeros_h, agg_h,
          idx_v, ba, acc):
        c = lax.axis_index("c")
        s = lax.axis_index("s")

        base = pl.multiple_of(s * stripe, 8)
        start = per * s + jnp.minimum(s, extra)
        count = per + (s < extra).astype(jnp.int32)
        row0 = c * E

        pltpu.sync_copy(zeros_h, acc.at[pl.ds(base, stripe)])
        plsc.subcore_barrier()

        def body(i, carry):
            t = start + i
            r = pl.multiple_of(t * _CH, _CH)
            pltpu.sync_copy(dst_h.at[pl.ds(r, _CH)], idx_v)
            pltpu.sync_copy(msgs_h.at[pl.ds(pl.multiple_of(row0 + r, _CH), _CH)], ba)
            pltpu.sync_copy(ba, acc.at[idx_v], add=True)
            return carry

        lax.fori_loop(0, count, body, 0)
        plsc.subcore_barrier()

        out_r = pl.multiple_of(c * n_pad + base, 8)
        pltpu.sync_copy(acc.at[pl.ds(base, stripe)],
                        agg_h.at[pl.ds(out_r, stripe)])

    agg = k(msgs.reshape(2 * E, 128), dst, jnp.zeros((stripe, 128), f32))
    return agg.reshape(2, n_pad, 128)[:, :_N_NODES, :]


def _unpack2(g):
    gw = jax.lax.bitcast_convert_type(g, jnp.uint32)
    lo = jax.lax.bitcast_convert_type(gw << 16, jnp.float32)
    hi = jax.lax.bitcast_convert_type(gw & jnp.uint32(0xFFFF0000), jnp.float32)
    return lo, hi


def _msg_kernel(
    gs_ref, gd_ref,
    eraw_ref,
    ee_whc_ref, ee_ws0_ref, ee_wsT_ref, ee_wsb_ref, ee_wvT_ref,
    m0_whT_ref, m0_wsT_ref, m0_wsb_ref, m0_wvT_ref,
    m1_whT_ref, m1_wsT_ref, m1_wsb_ref, m1_wvT_ref,
    m2_whT_ref, m2_wsT_ref, m2_wsb_ref, m2_wvT_ref,
    out_ref,
):
    e_s = eraw_ref[:, 0:1]
    whc = ee_whc_ref[0:1, :]
    vhx = eraw_ref[:, 1:2] * whc
    vhy = eraw_ref[:, 2:3] * whc
    vhz = eraw_ref[:, 3:4] * whc
    vn = _norm3(vhx, vhy, vhz)
    es = e_s * ee_ws0_ref[0:1, :] + _dot(vn, ee_wsT_ref[...]) + ee_wsb_ref[0:1, :]
    es = jnp.maximum(es, 0.0)
    evx = _dot(vhx, ee_wvT_ref[...])
    evy = _dot(vhy, ee_wvT_ref[...])
    evz = _dot(vhz, ee_wvT_ref[...])
    sg = jax.nn.sigmoid(_norm3(evx, evy, evz))
    evx, evy, evz = evx * sg, evy * sg, evz * sg

    lo_s, hi_s = _unpack2(gs_ref[...])
    lo_d, hi_d = _unpack2(gd_ref[...])
    s_src, vx_src = lo_s[:, 0:64], hi_s[:, 0:64]
    vy_src, vz_src = lo_s[:, 64:128], hi_s[:, 64:128]
    s_dst, vx_dst = lo_d[:, 0:64], hi_d[:, 0:64]
    vy_dst, vz_dst = lo_d[:, 64:128], hi_d[:, 64:128]
    whT = m0_whT_ref
    wA = whT[0:64, :]
    wB = whT[64:128, :]
    wC = whT[128:192, :]
    vhx = _dot(vx_src, wA) + _dot(evx, wB) + _dot(vx_dst, wC)
    vhy = _dot(vy_src, wA) + _dot(evy, wB) + _dot(vy_dst, wC)
    vhz = _dot(vz_src, wA) + _dot(evz, wB) + _dot(vz_dst, wC)
    vn = _norm3(vhx, vhy, vhz)
    wsT = m0_wsT_ref
    s = (_dot(s_src, wsT[0:64, :]) + _dot(es, wsT[64:128, :])
         + _dot(s_dst, wsT[128:192, :]) + _dot(vn, wsT[192:384, :])
         + m0_wsb_ref[0:1, :])
    vox = _dot(vhx, m0_wvT_ref[...])
    voy = _dot(vhy, m0_wvT_ref[...])
    voz = _dot(vhz, m0_wvT_ref[...])
    sg = jax.nn.sigmoid(_norm3(vox, voy, voz))
    vx, vy, vz = vox * sg, voy * sg, voz * sg
    s = jnp.maximum(s, 0.0)

    def gvp64(s, vx, vy, vz, whT, wsT, wsb, wvT, act):
        vhx = _dot(vx, whT[...])
        vhy = _dot(vy, whT[...])
        vhz = _dot(vz, whT[...])
        vn = _norm3(vhx, vhy, vhz)
        s_out = _dot(s, wsT[0:64, :]) + _dot(vn, wsT[64:128, :]) + wsb[0:1, :]
        vox = _dot(vhx, wvT[...])
        voy = _dot(vhy, wvT[...])
        voz = _dot(vhz, wvT[...])
        if act:
            sg = jax.nn.sigmoid(_norm3(vox, voy, voz))
            vox, voy, voz = vox * sg, voy * sg, voz * sg
            s_out = jnp.maximum(s_out, 0.0)
        return s_out, vox, voy, voz

    s, vx, vy, vz = gvp64(s, vx, vy, vz, m1_whT_ref, m1_wsT_ref, m1_wsb_ref, m1_wvT_ref, True)
    s, vx, vy, vz = gvp64(s, vx, vy, vz, m2_whT_ref, m2_wsT_ref, m2_wsb_ref, m2_wvT_ref, False)

    out_ref[0, :, 0:64] = s
    out_ref[0, :, 64:128] = vx
    out_ref[1, :, 0:64] = vy
    out_ref[1, :, 64:128] = vz


def _edge_messages(g, eraw, pe, p0, p1, p2):
    E = eraw.shape[0]
    B = _E_BLOCK
    nb = E // B
    grid = (nb,)
    f32 = jnp.float32

    def gspec(is_dst):
        return pl.BlockSpec(
            (B, 128),
            functools.partial(lambda o, i: (i + o, 0), nb if is_dst else 0))

    def wspec(shape):
        return pl.BlockSpec(shape, lambda i: tuple(0 for _ in shape))

    ee_whc = pe["wh"].T.reshape(1, 64)
    ee_ws0 = pe["ws"].T[0:1, :]
    ee_wsT = pe["ws"].T[1:, :]
    ee_wsb = pe["ws_b"].reshape(1, 64)
    ee_wvT = pe["wv"].T

    def prep(p):
        return (p["wh"].T, p["ws"].T, p["ws_b"].reshape(1, -1), p["wv"].T)

    m0, m1, m2 = prep(p0), prep(p1), prep(p2)

    in_arrays = [
        g, g,
        eraw,
        ee_whc, ee_ws0, ee_wsT, ee_wsb, ee_wvT,
        *m0, *m1, *m2,
    ]
    in_specs = [
        gspec(False), gspec(True),
        pl.BlockSpec((B, 4), lambda i: (i, 0)),
        wspec((1, 64)), wspec((1, 64)), wspec((64, 64)), wspec((1, 64)), wspec((64, 64)),
        wspec((192, 192)), wspec((384, 64)), wspec((1, 64)), wspec((192, 64)),
        wspec((64, 64)), wspec((128, 64)), wspec((1, 64)), wspec((64, 64)),
        wspec((64, 64)), wspec((128, 64)), wspec((1, 64)), wspec((64, 64)),
    ]
    return pl.pallas_call(
        _msg_kernel,
        grid=grid,
        in_specs=in_specs,
        out_specs=pl.BlockSpec((2, B, 128), lambda i: (0, i, 0)),
        out_shape=jax.ShapeDtypeStruct((2, E, 128), f32),
    )(*in_arrays)


def _gvp_t(p, s, vt, scalar_act, vector_act):
    vh = jnp.matmul(vt, p["wh"].T)
    vn = jnp.sqrt(jnp.maximum(jnp.sum(vh * vh, axis=-2), _EPS))
    s_out = jnp.matmul(jnp.concatenate([s, vn], axis=-1), p["ws"].T) + p["ws_b"]
    vt_out = jnp.matmul(vh, p["wv"].T)
    if vector_act:
        nrm = jnp.sqrt(jnp.maximum(jnp.sum(vt_out * vt_out, axis=-2, keepdims=True), _EPS))
        vt_out = vt_out * jax.nn.sigmoid(nrm)
    if scalar_act:
        s_out = jax.nn.relu(s_out)
    return s_out, vt_out


def _layernorm_s(g, b, s):
    mu = jnp.mean(s, axis=-1, keepdims=True)
    var = jnp.mean(jnp.square(s - mu), axis=-1, keepdims=True)
    return (s - mu) / jnp.sqrt(var + 1e-5) * g + b


def kernel(node_s, node_v, edge_index, edge_s, edge_v, batch, params):
    n = node_s.shape[0]
    num_graphs = 32
    if node_v.ndim == 2:
        node_v = node_v[:, None, :]
    if edge_v.ndim == 2:
        edge_v = edge_v[:, None, :]
    node_vt = jnp.swapaxes(node_v, -1, -2)
    ns, nvt = _gvp_t(params["node_enc"], node_s, node_vt, True, True)

    p = params["convs"][-1]
    E = edge_index.shape[1]

    def pack2(a, b):
        au = jax.lax.bitcast_convert_type(a.astype(jnp.bfloat16), jnp.uint16).astype(jnp.uint32)
        bu = jax.lax.bitcast_convert_type(b.astype(jnp.bfloat16), jnp.uint16).astype(jnp.uint32)
        return jax.lax.bitcast_convert_type(au | (bu << 16), jnp.float32)

    table = jnp.concatenate(
        [pack2(ns, nvt[:, 0, :]), pack2(nvt[:, 1, :], nvt[:, 2, :])], axis=1)
    eraw = jnp.concatenate([edge_s, edge_v[:, 0, :]], axis=-1)

    idx = edge_index.reshape(2 * E)
    g = _sc_gather(table, idx)

    msgs = _edge_messages(g, eraw, params["edge_enc"], p["msg0"], p["msg1"], p["msg2"])

    dst = edge_index[1]
    agg = _sc_scatter(msgs, dst)
    cnt = jnp.maximum(
        jax.ops.segment_sum(jnp.ones((E,), jnp.float32), dst, num_segments=n), 1.0)
    agg_s = agg[0, :, 0:64] / cnt[:, None]
    agg_v = jnp.stack([agg[0, :, 64:128], agg[1, :, 0:64], agg[1, :, 64:128]],
                      axis=1) / cnt[:, None, None]

    s1 = _layernorm_s(p["ln0_g"], p["ln0_b"], ns + agg_s)
    v1 = nvt + agg_v
    vn = jnp.maximum(jnp.sum(v1 * v1, axis=-2, keepdims=True), _EPS)
    vn = jnp.sqrt(jnp.mean(vn, axis=-1, keepdims=True))
    v1 = v1 / vn

    fs, fvt = _gvp_t(p["ff0"], s1, v1, True, True)
    fs, _ = _gvp_t(p["ff1"], fs, fvt, False, False)
    xs = _layernorm_s(p["ln1_g"], p["ln1_b"], s1 + fs)

    cntb = jnp.maximum(jax.ops.segment_sum(jnp.ones((n,), jnp.float32), batch, num_segments=num_graphs), 1.0)
    pooled = jax.ops.segment_sum(xs, batch, num_segments=num_graphs) / cntb[:, None]
    w0, b0 = params["mlp"][0]
    w1, b1 = params["mlp"][1]
    w2, b2 = params["mlp"][2]
    h = jax.nn.relu(jnp.matmul(pooled, w0.T) + b0)
    h = jax.nn.relu(jnp.matmul(h, w1.T) + b1)
    return jax.nn.sigmoid(jnp.matmul(h, w2.T) + b2)


# speedup vs baseline: 13.3726x; 1.0590x over previous
import functools

import jax
import jax.numpy as jnp
from jax import lax
from jax.experimental import pallas as pl
from jax.experimental.pallas import tpu as pltpu
from jax.experimental.pallas import tpu_sc as plsc

_EPS = 1e-8
_E_BLOCK = 1600
_CH = 128
_N_NODES = 10000
_NC, _NS = 2, 16


def _norm3(vx, vy, vz):
    return jnp.sqrt(jnp.maximum(vx * vx + vy * vy + vz * vz, _EPS))


def _dot(a, b):
    return jnp.dot(a, b, preferred_element_type=jnp.float32)


def _sc_gather(table, idx):
    rows = idx.shape[0]
    n_chunks = rows // _CH
    nw = _NC * _NS
    per, extra = divmod(n_chunks, nw)
    f32 = jnp.float32
    mesh = plsc.VectorSubcoreMesh(core_axis_name="c", subcore_axis_name="s")

    @functools.partial(
        pl.kernel,
        out_type=jax.ShapeDtypeStruct((rows, 128), f32),
        mesh=mesh,
        scratch_types=[
            pltpu.VMEM((_CH,), jnp.int32),
            pltpu.VMEM((_CH, 128), f32),
            pltpu.SemaphoreType.DMA,
        ],
    )
    def k(table_h, idx_h, out_h,
          idx_v, b0, s0):
        c = lax.axis_index("c")
        s = lax.axis_index("s")
        w = s * _NC + c
        start = per * w + jnp.minimum(w, extra)
        count = per + (w < extra).astype(jnp.int32)

        def body(i, carry):
            t = start + i
            r = pl.multiple_of(t * _CH, _CH)
            pltpu.sync_copy(idx_h.at[pl.ds(r, _CH)], idx_v)
            pltpu.async_copy(table_h.at[idx_v], b0, s0).wait()
            pltpu.sync_copy(b0, out_h.at[pl.ds(r, _CH)])
            return carry

        lax.fori_loop(0, count, body, 0)

    return k(table, idx)


def _sc_scatter(msgs, dst):
    E = dst.shape[0]
    n_chunks = E // _CH
    stripe = 632
    n_pad = stripe * _NS
    f32 = jnp.float32
    per, extra = divmod(n_chunks, _NS)
    mesh = plsc.VectorSubcoreMesh(core_axis_name="c", subcore_axis_name="s")

    @functools.partial(
        pl.kernel,
        out_type=jax.ShapeDtypeStruct((2 * n_pad, 128), f32),
        mesh=mesh,
        scratch_types=[
            pltpu.VMEM((_CH,), jnp.int32),
            pltpu.VMEM((_CH, 128), f32),
            pltpu.VMEM_SHARED((n_pad, 128), f32),
        ],
    )
    def k(msgs_h, dst_h, zeros_h, agg_h,
          idx_v, ba, acc):
        c = lax.axis_index("c")
        s = lax.axis_index("s")

        base = pl.multiple_of(s * stripe, 8)
        start = per * s + jnp.minimum(s, extra)
        count = per + (s < extra).astype(jnp.int32)
        row0 = c * E

        pltpu.sync_copy(zeros_h, acc.at[pl.ds(base, stripe)])
        plsc.subcore_barrier()

        def body(i, carry):
            t = start + i
            r = pl.multiple_of(t * _CH, _CH)
            pltpu.sync_copy(dst_h.at[pl.ds(r, _CH)], idx_v)
            pltpu.sync_copy(msgs_h.at[pl.ds(pl.multiple_of(row0 + r, _CH), _CH)], ba)
            pltpu.sync_copy(ba, acc.at[idx_v], add=True)
            return carry

        lax.fori_loop(0, count, body, 0)
        plsc.subcore_barrier()

        out_r = pl.multiple_of(c * n_pad + base, 8)
        pltpu.sync_copy(acc.at[pl.ds(base, stripe)],
                        agg_h.at[pl.ds(out_r, stripe)])

    agg = k(msgs.reshape(2 * E, 128), dst, jnp.zeros((stripe, 128), f32))
    return agg.reshape(2, n_pad, 128)[:, :_N_NODES, :]


def _unpack2(g):
    gw = jax.lax.bitcast_convert_type(g, jnp.uint32)
    lo = jax.lax.bitcast_convert_type(gw << 16, jnp.float32)
    hi = jax.lax.bitcast_convert_type(gw & jnp.uint32(0xFFFF0000), jnp.float32)
    return lo, hi


def _msg_kernel(
    gs_ref, gd_ref,
    eraw_ref,
    ee_whc_ref, ee_ws0_ref, ee_wsT_ref, ee_wsb_ref, ee_wvT_ref,
    m0_whT_ref, m0_wsT_ref, m0_wsb_ref, m0_wvT_ref,
    m1_whT_ref, m1_wsT_ref, m1_wsb_ref, m1_wvT_ref,
    m2_whT_ref, m2_wsT_ref, m2_wsb_ref, m2_wvT_ref,
    out_ref,
):
    e_s = eraw_ref[:, 0:1]
    whc = ee_whc_ref[0:1, :]
    vhx = eraw_ref[:, 1:2] * whc
    vhy = eraw_ref[:, 2:3] * whc
    vhz = eraw_ref[:, 3:4] * whc
    vn = _norm3(vhx, vhy, vhz)
    es = e_s * ee_ws0_ref[0:1, :] + _dot(vn, ee_wsT_ref[...]) + ee_wsb_ref[0:1, :]
    es = jnp.maximum(es, 0.0)
    evx = _dot(vhx, ee_wvT_ref[...])
    evy = _dot(vhy, ee_wvT_ref[...])
    evz = _dot(vhz, ee_wvT_ref[...])
    sg = jax.nn.sigmoid(_norm3(evx, evy, evz))
    evx, evy, evz = evx * sg, evy * sg, evz * sg

    lo_s, hi_s = _unpack2(gs_ref[...])
    lo_d, hi_d = _unpack2(gd_ref[...])
    s_src, vx_src = lo_s[:, 0:64], hi_s[:, 0:64]
    vy_src, vz_src = lo_s[:, 64:128], hi_s[:, 64:128]
    s_dst, vx_dst = lo_d[:, 0:64], hi_d[:, 0:64]
    vy_dst, vz_dst = lo_d[:, 64:128], hi_d[:, 64:128]
    whT = m0_whT_ref
    wA = whT[0:64, :]
    wB = whT[64:128, :]
    wC = whT[128:192, :]
    vhx = _dot(vx_src, wA) + _dot(evx, wB) + _dot(vx_dst, wC)
    vhy = _dot(vy_src, wA) + _dot(evy, wB) + _dot(vy_dst, wC)
    vhz = _dot(vz_src, wA) + _dot(evz, wB) + _dot(vz_dst, wC)
    vn = _norm3(vhx, vhy, vhz)
    wsT = m0_wsT_ref
    s = (_dot(s_src, wsT[0:64, :]) + _dot(es, wsT[64:128, :])
         + _dot(s_dst, wsT[128:192, :]) + _dot(vn, wsT[192:384, :])
         + m0_wsb_ref[0:1, :])
    vox = _dot(vhx, m0_wvT_ref[...])
    voy = _dot(vhy, m0_wvT_ref[...])
    voz = _dot(vhz, m0_wvT_ref[...])
    sg = jax.nn.sigmoid(_norm3(vox, voy, voz))
    vx, vy, vz = vox * sg, voy * sg, voz * sg
    s = jnp.maximum(s, 0.0)

    def gvp64(s, vx, vy, vz, whT, wsT, wsb, wvT, act):
        vhx = _dot(vx, whT[...])
        vhy = _dot(vy, whT[...])
        vhz = _dot(vz, whT[...])
        vn = _norm3(vhx, vhy, vhz)
        s_out = _dot(s, wsT[0:64, :]) + _dot(vn, wsT[64:128, :]) + wsb[0:1, :]
        vox = _dot(vhx, wvT[...])
        voy = _dot(vhy, wvT[...])
        voz = _dot(vhz, wvT[...])
        if act:
            sg = jax.nn.sigmoid(_norm3(vox, voy, voz))
            vox, voy, voz = vox * sg, voy * sg, voz * sg
            s_out = jnp.maximum(s_out, 0.0)
        return s_out, vox, voy, voz

    s, vx, vy, vz = gvp64(s, vx, vy, vz, m1_whT_ref, m1_wsT_ref, m1_wsb_ref, m1_wvT_ref, True)
    s, vx, vy, vz = gvp64(s, vx, vy, vz, m2_whT_ref, m2_wsT_ref, m2_wsb_ref, m2_wvT_ref, False)

    out_ref[0, :, 0:64] = s
    out_ref[0, :, 64:128] = vx
    out_ref[1, :, 0:64] = vy
    out_ref[1, :, 64:128] = vz


def _edge_messages(g, eraw, pe, p0, p1, p2):
    E = eraw.shape[0]
    B = _E_BLOCK
    nb = E // B
    grid = (nb,)
    f32 = jnp.float32

    def gspec(is_dst):
        return pl.BlockSpec(
            (B, 128),
            functools.partial(lambda o, i: (i + o, 0), nb if is_dst else 0))

    def wspec(shape):
        return pl.BlockSpec(shape, lambda i: tuple(0 for _ in shape))

    ee_whc = pe["wh"].T.reshape(1, 64)
    ee_ws0 = pe["ws"].T[0:1, :]
    ee_wsT = pe["ws"].T[1:, :]
    ee_wsb = pe["ws_b"].reshape(1, 64)
    ee_wvT = pe["wv"].T

    def prep(p):
        return (p["wh"].T, p["ws"].T, p["ws_b"].reshape(1, -1), p["wv"].T)

    m0, m1, m2 = prep(p0), prep(p1), prep(p2)

    in_arrays = [
        g, g,
        eraw,
        ee_whc, ee_ws0, ee_wsT, ee_wsb, ee_wvT,
        *m0, *m1, *m2,
    ]
    in_specs = [
        gspec(False), gspec(True),
        pl.BlockSpec((B, 4), lambda i: (i, 0)),
        wspec((1, 64)), wspec((1, 64)), wspec((64, 64)), wspec((1, 64)), wspec((64, 64)),
        wspec((192, 192)), wspec((384, 64)), wspec((1, 64)), wspec((192, 64)),
        wspec((64, 64)), wspec((128, 64)), wspec((1, 64)), wspec((64, 64)),
        wspec((64, 64)), wspec((128, 64)), wspec((1, 64)), wspec((64, 64)),
    ]
    return pl.pallas_call(
        _msg_kernel,
        grid=grid,
        in_specs=in_specs,
        out_specs=pl.BlockSpec((2, B, 128), lambda i: (0, i, 0)),
        out_shape=jax.ShapeDtypeStruct((2, E, 128), f32),
    )(*in_arrays)


def _pool_kernel(xs_ref, batch_ref, ps_ref, cb_ref):
    oh = (batch_ref[...] == jax.lax.broadcasted_iota(jnp.int32, (1, 32), 1)
          ).astype(jnp.float32)
    ps_ref[...] = jax.lax.dot_general(
        oh, xs_ref[...], (((0,), (0,)), ((), ())),
        preferred_element_type=jnp.float32)
    cb_ref[...] = jax.lax.dot_general(
        oh, jnp.ones_like(xs_ref[...][:, 0:8]), (((0,), (0,)), ((), ())),
        preferred_element_type=jnp.float32)


def _pool(xs, batch):
    n = xs.shape[0]
    ps, cb = pl.pallas_call(
        _pool_kernel,
        in_specs=[pl.BlockSpec((n, 64), lambda: (0, 0)),
                  pl.BlockSpec((n, 1), lambda: (0, 0))],
        out_specs=[pl.BlockSpec((32, 64), lambda: (0, 0)),
                   pl.BlockSpec((32, 8), lambda: (0, 0))],
        out_shape=[jax.ShapeDtypeStruct((32, 64), jnp.float32),
                   jax.ShapeDtypeStruct((32, 8), jnp.float32)],
    )(xs, batch.reshape(n, 1))
    return ps / jnp.maximum(cb[:, 0:1], 1.0)


def _gvp_t(p, s, vt, scalar_act, vector_act):
    vh = jnp.matmul(vt, p["wh"].T)
    vn = jnp.sqrt(jnp.maximum(jnp.sum(vh * vh, axis=-2), _EPS))
    s_out = jnp.matmul(jnp.concatenate([s, vn], axis=-1), p["ws"].T) + p["ws_b"]
    vt_out = jnp.matmul(vh, p["wv"].T)
    if vector_act:
        nrm = jnp.sqrt(jnp.maximum(jnp.sum(vt_out * vt_out, axis=-2, keepdims=True), _EPS))
        vt_out = vt_out * jax.nn.sigmoid(nrm)
    if scalar_act:
        s_out = jax.nn.relu(s_out)
    return s_out, vt_out


def _layernorm_s(g, b, s):
    mu = jnp.mean(s, axis=-1, keepdims=True)
    var = jnp.mean(jnp.square(s - mu), axis=-1, keepdims=True)
    return (s - mu) / jnp.sqrt(var + 1e-5) * g + b


def kernel(node_s, node_v, edge_index, edge_s, edge_v, batch, params):
    n = node_s.shape[0]
    num_graphs = 32
    if node_v.ndim == 2:
        node_v = node_v[:, None, :]
    if edge_v.ndim == 2:
        edge_v = edge_v[:, None, :]
    node_vt = jnp.swapaxes(node_v, -1, -2)
    ns, nvt = _gvp_t(params["node_enc"], node_s, node_vt, True, True)

    p = params["convs"][-1]
    E = edge_index.shape[1]

    def pack2(a, b):
        au = jax.lax.bitcast_convert_type(a.astype(jnp.bfloat16), jnp.uint16).astype(jnp.uint32)
        bu = jax.lax.bitcast_convert_type(b.astype(jnp.bfloat16), jnp.uint16).astype(jnp.uint32)
        return jax.lax.bitcast_convert_type(au | (bu << 16), jnp.float32)

    table = jnp.concatenate(
        [pack2(ns, nvt[:, 0, :]), pack2(nvt[:, 1, :], nvt[:, 2, :])], axis=1)
    eraw = jnp.concatenate([edge_s, edge_v[:, 0, :]], axis=-1)

    idx = edge_index.reshape(2 * E)
    g = _sc_gather(table, idx)

    msgs = _edge_messages(g, eraw, params["edge_enc"], p["msg0"], p["msg1"], p["msg2"])

    dst = edge_index[1]
    agg = _sc_scatter(msgs, dst)
    cnt = jnp.maximum(
        jax.ops.segment_sum(jnp.ones((E,), jnp.float32), dst, num_segments=n), 1.0)
    agg_s = agg[0, :, 0:64] / cnt[:, None]
    agg_v = jnp.stack([agg[0, :, 64:128], agg[1, :, 0:64], agg[1, :, 64:128]],
                      axis=1) / cnt[:, None, None]

    s1 = _layernorm_s(p["ln0_g"], p["ln0_b"], ns + agg_s)
    v1 = nvt + agg_v
    vn = jnp.maximum(jnp.sum(v1 * v1, axis=-2, keepdims=True), _EPS)
    vn = jnp.sqrt(jnp.mean(vn, axis=-1, keepdims=True))
    v1 = v1 / vn

    fs, fvt = _gvp_t(p["ff0"], s1, v1, True, True)
    fs, _ = _gvp_t(p["ff1"], fs, fvt, False, False)
    xs = _layernorm_s(p["ln1_g"], p["ln1_b"], s1 + fs)

    pooled = _pool(xs, batch)
    w0, b0 = params["mlp"][0]
    w1, b1 = params["mlp"][1]
    w2, b2 = params["mlp"][2]
    h = jax.nn.relu(jnp.matmul(pooled, w0.T) + b0)
    h = jax.nn.relu(jnp.matmul(h, w1.T) + b1)
    return jax.nn.sigmoid(jnp.matmul(h, w2.T) + b2)
